# baseline (speedup 1.0000x reference)
import jax
import jax.numpy as jnp
from jax import lax
from jax.experimental import pallas as pl
from jax.experimental.pallas import tpu as pltpu
from jax.experimental.pallas import tpu_sc as plsc

VOCAB = 1000000
EMBED_DIM = 32
BATCH = 4096
SEQ_LEN = 50
L_TOT = 3 * SEQ_LEN
BT = BATCH // 128
NJOBS = L_TOT * BT

NUM_CORES = 2
NUM_SUBCORES = 16
NUM_WORKERS = NUM_CORES * NUM_SUBCORES
JOBS_PER_W = NJOBS // NUM_WORKERS
IDX_PER_W = JOBS_PER_W * 128


def _body(idx_ref, table_ref, out_ref,
          idx_v, rows_a, rows_b, tb_a, tb_b,
          gsem_a, gsem_b, osem_a, osem_b):
    wid = lax.axis_index("s") * NUM_CORES + lax.axis_index("c")
    job0 = wid * JOBS_PER_W

    pltpu.sync_copy(idx_ref.at[pl.ds(job0 * 128, IDX_PER_W)], idx_v)

    iota16 = lax.iota(jnp.int32, 16)
    row_idx = [iota16 + 16 * c0 for c0 in range(8)]

    def issue_gather(jloc, rows, sem):
        pltpu.async_copy(table_ref.at[idx_v.at[pl.ds(jloc * 128, 128)]],
                         rows, sem)

    def wait_gather(jloc, rows, sem):
        pltpu.make_async_copy(table_ref.at[idx_v.at[pl.ds(jloc * 128, 128)]],
                              rows, sem).wait()

    def transpose(rows, tb):
        for d in range(EMBED_DIM):
            dv = jnp.full((16,), d, dtype=jnp.int32)
            for c0 in range(8):
                v = plsc.load_gather(rows, [row_idx[c0], dv])
                tb[d, pl.ds(16 * c0, 16)] = v

    def issue_out(jloc, tb, sem):
        job = job0 + jloc
        lt = (job // BT) * 128 + (job % BT)
        for a in range(4):
            pltpu.async_copy(tb.at[pl.ds(8 * a, 8)],
                             out_ref.at[pl.ds((lt + 32 * a) * 8, 8)], sem)

    def wait_out(jloc, tb, sem):
        job = job0 + jloc
        lt = (job // BT) * 128 + (job % BT)
        for a in range(4):
            pltpu.make_async_copy(tb.at[pl.ds(8 * a, 8)],
                                  out_ref.at[pl.ds((lt + 32 * a) * 8, 8)],
                                  sem).wait()

    issue_gather(0, rows_a, gsem_a)

    half = JOBS_PER_W // 2

    def step(i, _):
        ja = 2 * i
        jb = 2 * i + 1
        issue_gather(jb, rows_b, gsem_b)
        wait_gather(ja, rows_a, gsem_a)

        @pl.when(i >= 1)
        def _():
            wait_out(ja - 2, tb_a, osem_a)
        transpose(rows_a, tb_a)
        issue_out(ja, tb_a, osem_a)
        @pl.when(i <= half - 2)
        def _():
            issue_gather(jb + 1, rows_a, gsem_a)
        wait_gather(jb, rows_b, gsem_b)

        @pl.when(i >= 1)
        def _():
            wait_out(jb - 2, tb_b, osem_b)
        transpose(rows_b, tb_b)
        issue_out(jb, tb_b, osem_b)
        return 0

    lax.fori_loop(0, half, step, 0)
    wait_out(JOBS_PER_W - 2, tb_a, osem_a)
    wait_out(JOBS_PER_W - 1, tb_b, osem_b)


_NSUP = (VOCAB + 2047) // 2048
_VPAD = _NSUP * 2048


def _tc_linearize(table):
    def body(in_ref, out_ref):
        for q in range(4):
            out_ref[:, 32 * q:32 * (q + 1)] = in_ref[:, 512 * q:512 * (q + 1)].T

    y = pl.pallas_call(
        body,
        grid=(_NSUP,),
        in_specs=[pl.BlockSpec((EMBED_DIM, 2048), lambda i: (0, i))],
        out_specs=pl.BlockSpec((512, 128), lambda i: (i, 0)),
        out_shape=jax.ShapeDtypeStruct((_VPAD // 4, 128), jnp.float32),
    )(table.T)
    return y.reshape(_VPAD, EMBED_DIM)


@jax.jit
def kernel(c1_idx, c2_idx, c3_idx, table):
    v = jnp.concatenate(
        [c1_idx.T.astype(jnp.int32), c2_idx.T.astype(jnp.int32),
         c3_idx.T.astype(jnp.int32)], axis=0).reshape(NJOBS * 128)
    idx_flat = ((v >> 11) << 11) | ((v & 511) << 2) | ((v >> 9) & 3)

    mesh = plsc.VectorSubcoreMesh(
        core_axis_name="c", subcore_axis_name="s",
        num_cores=NUM_CORES, num_subcores=NUM_SUBCORES)
    out2 = pl.kernel(
        _body,
        out_type=jax.ShapeDtypeStruct((NJOBS * 32, 128), jnp.float32),
        mesh=mesh,
        scratch_types=[
            pltpu.VMEM((IDX_PER_W,), jnp.int32),
            pltpu.VMEM((128, EMBED_DIM), jnp.float32),
            pltpu.VMEM((128, EMBED_DIM), jnp.float32),
            pltpu.VMEM((EMBED_DIM, 128), jnp.float32),
            pltpu.VMEM((EMBED_DIM, 128), jnp.float32),
            pltpu.SemaphoreType.DMA,
            pltpu.SemaphoreType.DMA,
            pltpu.SemaphoreType.DMA,
            pltpu.SemaphoreType.DMA,
        ],
        compiler_params=pltpu.CompilerParams(use_tc_tiling_on_sc=False,
                                             needs_layout_passes=False),
    )(idx_flat, _tc_linearize(table))

    return (out2.reshape(L_TOT, 4, BT, 8, 128)
            .transpose(2, 4, 1, 3, 0)
            .reshape(BATCH, EMBED_DIM, L_TOT))


# speedup vs baseline: 1.1820x; 1.1820x over previous
import jax
import jax.numpy as jnp
from jax import lax
from jax.experimental import pallas as pl
from jax.experimental.pallas import tpu as pltpu
from jax.experimental.pallas import tpu_sc as plsc

VOCAB = 1000000
EMBED_DIM = 32
BATCH = 4096
SEQ_LEN = 50
L_TOT = 3 * SEQ_LEN
BT = BATCH // 128
NJOBS = L_TOT * BT

NUM_CORES = 2
NUM_SUBCORES = 16
NUM_WORKERS = NUM_CORES * NUM_SUBCORES
JOBS_PER_W = NJOBS // NUM_WORKERS
IDX_PER_W = JOBS_PER_W * 128


def _body(idx_ref, table_ref, out_ref,
          idx_v, rows_a, rows_b, tb_a, tb_b,
          gsem_a, gsem_b, osem_a, osem_b):
    wid = lax.axis_index("s") * NUM_CORES + lax.axis_index("c")
    job0 = wid * JOBS_PER_W

    pltpu.sync_copy(idx_ref.at[pl.ds(job0 * 128, IDX_PER_W)], idx_v)

    iota_lo = lax.iota(jnp.int32, 16)
    iota_hi = iota_lo + 16

    def issue_gather(jloc, rows, sem):
        pltpu.async_copy(table_ref.at[idx_v.at[pl.ds(jloc * 128, 128)]],
                         rows, sem)

    def wait_gather(jloc, rows, sem):
        pltpu.make_async_copy(table_ref.at[idx_v.at[pl.ds(jloc * 128, 128)]],
                              rows, sem).wait()

    def transpose(rows, tb):
        for r in range(128):
            rv = jnp.full((16,), r, dtype=jnp.int32)
            plsc.store_scatter(tb, [iota_lo, rv], rows[r, 0:16])
            plsc.store_scatter(tb, [iota_hi, rv], rows[r, 16:32])

    def issue_out(jloc, tb, sem):
        job = job0 + jloc
        lt = (job // BT) * 128 + (job % BT)
        for a in range(4):
            pltpu.async_copy(tb.at[pl.ds(8 * a, 8)],
                             out_ref.at[pl.ds((lt + 32 * a) * 8, 8)], sem)

    def wait_out(jloc, tb, sem):
        job = job0 + jloc
        lt = (job // BT) * 128 + (job % BT)
        for a in range(4):
            pltpu.make_async_copy(tb.at[pl.ds(8 * a, 8)],
                                  out_ref.at[pl.ds((lt + 32 * a) * 8, 8)],
                                  sem).wait()

    issue_gather(0, rows_a, gsem_a)

    half = JOBS_PER_W // 2

    def step(i, _):
        ja = 2 * i
        jb = 2 * i + 1
        issue_gather(jb, rows_b, gsem_b)
        wait_gather(ja, rows_a, gsem_a)

        @pl.when(i >= 1)
        def _():
            wait_out(ja - 2, tb_a, osem_a)
        transpose(rows_a, tb_a)
        issue_out(ja, tb_a, osem_a)
        @pl.when(i <= half - 2)
        def _():
            issue_gather(jb + 1, rows_a, gsem_a)
        wait_gather(jb, rows_b, gsem_b)

        @pl.when(i >= 1)
        def _():
            wait_out(jb - 2, tb_b, osem_b)
        transpose(rows_b, tb_b)
        issue_out(jb, tb_b, osem_b)
        return 0

    lax.fori_loop(0, half, step, 0)
    wait_out(JOBS_PER_W - 2, tb_a, osem_a)
    wait_out(JOBS_PER_W - 1, tb_b, osem_b)


_NSUP = (VOCAB + 2047) // 2048
_VPAD = _NSUP * 2048


def _tc_linearize(table):
    def body(in_ref, out_ref):
        for q in range(4):
            out_ref[:, 32 * q:32 * (q + 1)] = in_ref[:, 512 * q:512 * (q + 1)].T

    y = pl.pallas_call(
        body,
        grid=(_NSUP,),
        in_specs=[pl.BlockSpec((EMBED_DIM, 2048), lambda i: (0, i))],
        out_specs=pl.BlockSpec((512, 128), lambda i: (i, 0)),
        out_shape=jax.ShapeDtypeStruct((_VPAD // 4, 128), jnp.float32),
    )(table.T)
    return y.reshape(_VPAD, EMBED_DIM)


@jax.jit
def kernel(c1_idx, c2_idx, c3_idx, table):
    v = jnp.concatenate(
        [c1_idx.T.astype(jnp.int32), c2_idx.T.astype(jnp.int32),
         c3_idx.T.astype(jnp.int32)], axis=0).reshape(NJOBS * 128)
    idx_flat = ((v >> 11) << 11) | ((v & 511) << 2) | ((v >> 9) & 3)

    mesh = plsc.VectorSubcoreMesh(
        core_axis_name="c", subcore_axis_name="s",
        num_cores=NUM_CORES, num_subcores=NUM_SUBCORES)
    out2 = pl.kernel(
        _body,
        out_type=jax.ShapeDtypeStruct((NJOBS * 32, 128), jnp.float32),
        mesh=mesh,
        scratch_types=[
            pltpu.VMEM((IDX_PER_W,), jnp.int32),
            pltpu.VMEM((128, EMBED_DIM), jnp.float32),
            pltpu.VMEM((128, EMBED_DIM), jnp.float32),
            pltpu.VMEM((EMBED_DIM, 128), jnp.float32),
            pltpu.VMEM((EMBED_DIM, 128), jnp.float32),
            pltpu.SemaphoreType.DMA,
            pltpu.SemaphoreType.DMA,
            pltpu.SemaphoreType.DMA,
            pltpu.SemaphoreType.DMA,
        ],
        compiler_params=pltpu.CompilerParams(use_tc_tiling_on_sc=False,
                                             needs_layout_passes=False),
    )(idx_flat, _tc_linearize(table))

    return (out2.reshape(L_TOT, 4, BT, 8, 128)
            .transpose(2, 4, 1, 3, 0)
            .reshape(BATCH, EMBED_DIM, L_TOT))


# speedup vs baseline: 1.6181x; 1.3690x over previous
import jax
import jax.numpy as jnp
from jax import lax
from jax.experimental import pallas as pl
from jax.experimental.pallas import tpu as pltpu
from jax.experimental.pallas import tpu_sc as plsc

VOCAB = 1000000
EMBED_DIM = 32
BATCH = 4096
SEQ_LEN = 50
L_TOT = 3 * SEQ_LEN
BT = BATCH // 128
NJOBS = L_TOT * BT

NUM_CORES = 2
NUM_SUBCORES = 16
NUM_WORKERS = NUM_CORES * NUM_SUBCORES
JOBS_PER_W = NJOBS // NUM_WORKERS
IDX_PER_W = JOBS_PER_W * 128


def _body(idx_ref, table_ref, out_ref,
          idx_v, rows_a, rows_b, tb_a, tb_b,
          gsem_a, gsem_b, osem_a, osem_b):
    wid = lax.axis_index("s") * NUM_CORES + lax.axis_index("c")
    job0 = wid * JOBS_PER_W

    pltpu.sync_copy(idx_ref.at[pl.ds(job0 * 128, IDX_PER_W)], idx_v)

    iota_lo = lax.iota(jnp.int32, 16)
    iota_hi = iota_lo + 16

    def issue_gather(jloc, rows, sem):
        pltpu.async_copy(table_ref.at[idx_v.at[pl.ds(jloc * 128, 128)]],
                         rows, sem)

    def wait_gather(jloc, rows, sem):
        pltpu.make_async_copy(table_ref.at[idx_v.at[pl.ds(jloc * 128, 128)]],
                              rows, sem).wait()

    def transpose(rows, tb):
        for r in range(128):
            rv = jnp.full((16,), r, dtype=jnp.int32)
            plsc.store_scatter(tb, [iota_lo, rv], rows[r, 0:16])
            plsc.store_scatter(tb, [iota_hi, rv], rows[r, 16:32])

    def issue_out(jloc, tb, sem):
        job = job0 + jloc
        lt = (job // BT) * 128 + (job % BT)
        for a in range(4):
            pltpu.async_copy(tb.at[pl.ds(8 * a, 8), pl.ds(0, 128)],
                             out_ref.at[pl.ds((lt + 32 * a) * 8, 8)], sem)

    def wait_out(jloc, tb, sem):
        job = job0 + jloc
        lt = (job // BT) * 128 + (job % BT)
        for a in range(4):
            pltpu.make_async_copy(tb.at[pl.ds(8 * a, 8), pl.ds(0, 128)],
                                  out_ref.at[pl.ds((lt + 32 * a) * 8, 8)],
                                  sem).wait()

    issue_gather(0, rows_a, gsem_a)

    half = JOBS_PER_W // 2

    def step(i, _):
        ja = 2 * i
        jb = 2 * i + 1
        issue_gather(jb, rows_b, gsem_b)
        wait_gather(ja, rows_a, gsem_a)

        @pl.when(i >= 1)
        def _():
            wait_out(ja - 2, tb_a, osem_a)
        transpose(rows_a, tb_a)
        issue_out(ja, tb_a, osem_a)
        @pl.when(i <= half - 2)
        def _():
            issue_gather(jb + 1, rows_a, gsem_a)
        wait_gather(jb, rows_b, gsem_b)

        @pl.when(i >= 1)
        def _():
            wait_out(jb - 2, tb_b, osem_b)
        transpose(rows_b, tb_b)
        issue_out(jb, tb_b, osem_b)
        return 0

    lax.fori_loop(0, half, step, 0)
    wait_out(JOBS_PER_W - 2, tb_a, osem_a)
    wait_out(JOBS_PER_W - 1, tb_b, osem_b)


_NSUP = (VOCAB + 2047) // 2048
_VPAD = _NSUP * 2048


def _tc_linearize(table):
    def body(in_ref, out_ref):
        for q in range(4):
            out_ref[:, 32 * q:32 * (q + 1)] = in_ref[:, 512 * q:512 * (q + 1)].T

    y = pl.pallas_call(
        body,
        grid=(_NSUP,),
        in_specs=[pl.BlockSpec((EMBED_DIM, 2048), lambda i: (0, i))],
        out_specs=pl.BlockSpec((512, 128), lambda i: (i, 0)),
        out_shape=jax.ShapeDtypeStruct((_VPAD // 4, 128), jnp.float32),
    )(table.T)
    return y.reshape(_VPAD, EMBED_DIM)


@jax.jit
def kernel(c1_idx, c2_idx, c3_idx, table):
    v = jnp.concatenate(
        [c1_idx.T.astype(jnp.int32), c2_idx.T.astype(jnp.int32),
         c3_idx.T.astype(jnp.int32)], axis=0).reshape(NJOBS * 128)
    idx_flat = ((v >> 11) << 11) | ((v & 511) << 2) | ((v >> 9) & 3)

    mesh = plsc.VectorSubcoreMesh(
        core_axis_name="c", subcore_axis_name="s",
        num_cores=NUM_CORES, num_subcores=NUM_SUBCORES)
    out2 = pl.kernel(
        _body,
        out_type=jax.ShapeDtypeStruct((NJOBS * 32, 128), jnp.float32),
        mesh=mesh,
        scratch_types=[
            pltpu.VMEM((IDX_PER_W,), jnp.int32),
            pltpu.VMEM((128, EMBED_DIM), jnp.float32),
            pltpu.VMEM((128, EMBED_DIM), jnp.float32),
            pltpu.VMEM((EMBED_DIM, 129), jnp.float32),
            pltpu.VMEM((EMBED_DIM, 129), jnp.float32),
            pltpu.SemaphoreType.DMA,
            pltpu.SemaphoreType.DMA,
            pltpu.SemaphoreType.DMA,
            pltpu.SemaphoreType.DMA,
        ],
        compiler_params=pltpu.CompilerParams(use_tc_tiling_on_sc=False,
                                             needs_layout_passes=False),
    )(idx_flat, _tc_linearize(table))

    return (out2.reshape(L_TOT, 4, BT, 8, 128)
            .transpose(2, 4, 1, 3, 0)
            .reshape(BATCH, EMBED_DIM, L_TOT))


# speedup vs baseline: 2.2217x; 1.3731x over previous
import jax
import jax.numpy as jnp
from jax import lax
from jax.experimental import pallas as pl
from jax.experimental.pallas import tpu as pltpu
from jax.experimental.pallas import tpu_sc as plsc

VOCAB = 1000000
EMBED_DIM = 32
BATCH = 4096
SEQ_LEN = 50
L_TOT = 3 * SEQ_LEN
BT = BATCH // 128
NJOBS = L_TOT * BT

NUM_CORES = 2
NUM_SUBCORES = 16
NUM_WORKERS = NUM_CORES * NUM_SUBCORES
JOBS_PER_W = NJOBS // NUM_WORKERS
IDX_PER_W = JOBS_PER_W * 128


def _body(idx_ref, table_ref, out_ref,
          idx_v, rows_a, rows_b, tb_a, tb_b,
          gsem_a, gsem_b, osem_a, osem_b):
    wid = lax.axis_index("s") * NUM_CORES + lax.axis_index("c")
    job0 = wid * JOBS_PER_W

    pltpu.sync_copy(idx_ref.at[pl.ds(job0 * 128, IDX_PER_W)], idx_v)

    iota_lo = lax.iota(jnp.int32, 16)
    iota_hi = iota_lo + 16

    def issue_gather(jloc, rows, sem):
        pltpu.async_copy(table_ref.at[idx_v.at[pl.ds(jloc * 128, 128)]],
                         rows, sem)

    def wait_gather(jloc, rows, sem):
        pltpu.make_async_copy(table_ref.at[idx_v.at[pl.ds(jloc * 128, 128)]],
                              rows, sem).wait()

    def transpose(rows, tb):
        for r in range(128):
            rv = jnp.full((16,), r, dtype=jnp.int32)
            plsc.store_scatter(tb, [iota_lo, rv], rows[r, 0:16])
            plsc.store_scatter(tb, [iota_hi, rv], rows[r, 16:32])

    def issue_out(jloc, tb, sem):
        job = job0 + jloc
        lt = (job // BT) * 128 + (job % BT)
        for a in range(4):
            pltpu.async_copy(tb.at[pl.ds(8 * a, 8), pl.ds(0, 128)],
                             out_ref.at[pl.ds((lt + 32 * a) * 8, 8)], sem)

    def wait_out(jloc, tb, sem):
        job = job0 + jloc
        lt = (job // BT) * 128 + (job % BT)
        for a in range(4):
            pltpu.make_async_copy(tb.at[pl.ds(8 * a, 8), pl.ds(0, 128)],
                                  out_ref.at[pl.ds((lt + 32 * a) * 8, 8)],
                                  sem).wait()

    issue_gather(0, rows_a, gsem_a)

    half = JOBS_PER_W // 2

    def step(i, _):
        ja = 2 * i
        jb = 2 * i + 1
        issue_gather(jb, rows_b, gsem_b)
        wait_gather(ja, rows_a, gsem_a)

        @pl.when(i >= 1)
        def _():
            wait_out(ja - 2, tb_a, osem_a)
        transpose(rows_a, tb_a)
        issue_out(ja, tb_a, osem_a)
        @pl.when(i <= half - 2)
        def _():
            issue_gather(jb + 1, rows_a, gsem_a)
        wait_gather(jb, rows_b, gsem_b)

        @pl.when(i >= 1)
        def _():
            wait_out(jb - 2, tb_b, osem_b)
        transpose(rows_b, tb_b)
        issue_out(jb, tb_b, osem_b)
        return 0

    lax.fori_loop(0, half, step, 0)
    wait_out(JOBS_PER_W - 2, tb_a, osem_a)
    wait_out(JOBS_PER_W - 1, tb_b, osem_b)


_NSUP = (VOCAB + 2047) // 2048
_VPAD = _NSUP * 2048


def _tc_linearize(table):
    def body(in_ref, out_ref):
        for q in range(16):
            out_ref[512 * (q // 4):512 * (q // 4 + 1), 32 * (q % 4):32 * (q % 4 + 1)] = in_ref[:, 512 * q:512 * (q + 1)].T

    y = pl.pallas_call(
        body,
        grid=((_VPAD + 8191) // 8192,),
        in_specs=[pl.BlockSpec((EMBED_DIM, 8192), lambda i: (0, i))],
        out_specs=pl.BlockSpec((2048, 128), lambda i: (i, 0)),
        out_shape=jax.ShapeDtypeStruct((_VPAD // 4, 128), jnp.float32),
    )(table.T)
    return y.reshape(_VPAD, EMBED_DIM)


@jax.jit
def kernel(c1_idx, c2_idx, c3_idx, table):
    v = jnp.concatenate(
        [c1_idx.T.astype(jnp.int32), c2_idx.T.astype(jnp.int32),
         c3_idx.T.astype(jnp.int32)], axis=0).reshape(NJOBS * 128)
    idx_flat = ((v >> 11) << 11) | ((v & 511) << 2) | ((v >> 9) & 3)

    mesh = plsc.VectorSubcoreMesh(
        core_axis_name="c", subcore_axis_name="s",
        num_cores=NUM_CORES, num_subcores=NUM_SUBCORES)
    out2 = pl.kernel(
        _body,
        out_type=jax.ShapeDtypeStruct((NJOBS * 32, 128), jnp.float32),
        mesh=mesh,
        scratch_types=[
            pltpu.VMEM((IDX_PER_W,), jnp.int32),
            pltpu.VMEM((128, EMBED_DIM), jnp.float32),
            pltpu.VMEM((128, EMBED_DIM), jnp.float32),
            pltpu.VMEM((EMBED_DIM, 129), jnp.float32),
            pltpu.VMEM((EMBED_DIM, 129), jnp.float32),
            pltpu.SemaphoreType.DMA,
            pltpu.SemaphoreType.DMA,
            pltpu.SemaphoreType.DMA,
            pltpu.SemaphoreType.DMA,
        ],
        compiler_params=pltpu.CompilerParams(use_tc_tiling_on_sc=False,
                                             needs_layout_passes=False),
    )(idx_flat, _tc_linearize(table))

    return (out2.reshape(L_TOT, 4, BT, 8, 128)
            .transpose(2, 4, 1, 3, 0)
            .reshape(BATCH, EMBED_DIM, L_TOT))


# speedup vs baseline: 2.2427x; 1.0094x over previous
import jax
import jax.numpy as jnp
from jax import lax
from jax.experimental import pallas as pl
from jax.experimental.pallas import tpu as pltpu
from jax.experimental.pallas import tpu_sc as plsc

VOCAB = 1000000
EMBED_DIM = 32
BATCH = 4096
SEQ_LEN = 50
L_TOT = 3 * SEQ_LEN
BT = BATCH // 128
NJOBS = L_TOT * BT

NUM_CORES = 2
NUM_SUBCORES = 16
NUM_WORKERS = NUM_CORES * NUM_SUBCORES
JOBS_PER_W = NJOBS // NUM_WORKERS
IDX_PER_W = JOBS_PER_W * 128


def _body(idx_ref, table_ref, out_ref,
          idx_v, rows_a, rows_b, tb_a, tb_b,
          gsem_a, gsem_b, osem_a, osem_b):
    wid = lax.axis_index("s") * NUM_CORES + lax.axis_index("c")
    job0 = wid * JOBS_PER_W

    pltpu.sync_copy(idx_ref.at[pl.ds(job0 * 128, IDX_PER_W)], idx_v)

    iota_lo = lax.iota(jnp.int32, 16)
    iota_hi = iota_lo + 16

    def issue_gather(jloc, rows, sem):
        pltpu.async_copy(table_ref.at[idx_v.at[pl.ds(jloc * 128, 128)]],
                         rows, sem)

    def wait_gather(jloc, rows, sem):
        pltpu.make_async_copy(table_ref.at[idx_v.at[pl.ds(jloc * 128, 128)]],
                              rows, sem).wait()

    def transpose(rows, tb):
        for r in range(128):
            rv = jnp.full((16,), r, dtype=jnp.int32)
            plsc.store_scatter(tb, [iota_lo, rv], rows[r, 0:16])
            plsc.store_scatter(tb, [iota_hi, rv], rows[r, 16:32])

    def issue_out(jloc, tb, sem):
        job = job0 + jloc
        lt = (job // BT) * 128 + (job % BT)
        for a in range(4):
            pltpu.async_copy(tb.at[pl.ds(8 * a, 8), pl.ds(0, 128)],
                             out_ref.at[pl.ds((lt + 32 * a) * 8, 8)], sem)

    def wait_out(jloc, tb, sem):
        job = job0 + jloc
        lt = (job // BT) * 128 + (job % BT)
        for a in range(4):
            pltpu.make_async_copy(tb.at[pl.ds(8 * a, 8), pl.ds(0, 128)],
                                  out_ref.at[pl.ds((lt + 32 * a) * 8, 8)],
                                  sem).wait()

    issue_gather(0, rows_a, gsem_a)

    half = JOBS_PER_W // 2

    def step(i, _):
        ja = 2 * i
        jb = 2 * i + 1
        issue_gather(jb, rows_b, gsem_b)
        wait_gather(ja, rows_a, gsem_a)

        @pl.when(i >= 1)
        def _():
            wait_out(ja - 2, tb_a, osem_a)
        transpose(rows_a, tb_a)
        issue_out(ja, tb_a, osem_a)
        @pl.when(i <= half - 2)
        def _():
            issue_gather(jb + 1, rows_a, gsem_a)
        wait_gather(jb, rows_b, gsem_b)

        @pl.when(i >= 1)
        def _():
            wait_out(jb - 2, tb_b, osem_b)
        transpose(rows_b, tb_b)
        issue_out(jb, tb_b, osem_b)
        return 0

    lax.fori_loop(0, half, step, 0)
    wait_out(JOBS_PER_W - 2, tb_a, osem_a)
    wait_out(JOBS_PER_W - 1, tb_b, osem_b)


_NSUP = (VOCAB + 2047) // 2048
_VPAD = _NSUP * 2048


def _tc_linearize(table):
    def body(in_ref, out_ref):
        for q in range(32):
            out_ref[512 * (q // 4):512 * (q // 4 + 1), 32 * (q % 4):32 * (q % 4 + 1)] = in_ref[:, 512 * q:512 * (q + 1)].T

    y = pl.pallas_call(
        body,
        grid=((_VPAD + 16383) // 16384,),
        in_specs=[pl.BlockSpec((EMBED_DIM, 16384), lambda i: (0, i))],
        out_specs=pl.BlockSpec((4096, 128), lambda i: (i, 0)),
        out_shape=jax.ShapeDtypeStruct((_VPAD // 4, 128), jnp.float32),
    )(table.T)
    return y.reshape(_VPAD, EMBED_DIM)


@jax.jit
def kernel(c1_idx, c2_idx, c3_idx, table):
    v = jnp.concatenate(
        [c1_idx.T.astype(jnp.int32), c2_idx.T.astype(jnp.int32),
         c3_idx.T.astype(jnp.int32)], axis=0).reshape(NJOBS * 128)
    idx_flat = ((v >> 11) << 11) | ((v & 511) << 2) | ((v >> 9) & 3)

    mesh = plsc.VectorSubcoreMesh(
        core_axis_name="c", subcore_axis_name="s",
        num_cores=NUM_CORES, num_subcores=NUM_SUBCORES)
    out2 = pl.kernel(
        _body,
        out_type=jax.ShapeDtypeStruct((NJOBS * 32, 128), jnp.float32),
        mesh=mesh,
        scratch_types=[
            pltpu.VMEM((IDX_PER_W,), jnp.int32),
            pltpu.VMEM((128, EMBED_DIM), jnp.float32),
            pltpu.VMEM((128, EMBED_DIM), jnp.float32),
            pltpu.VMEM((EMBED_DIM, 129), jnp.float32),
            pltpu.VMEM((EMBED_DIM, 129), jnp.float32),
            pltpu.SemaphoreType.DMA,
            pltpu.SemaphoreType.DMA,
            pltpu.SemaphoreType.DMA,
            pltpu.SemaphoreType.DMA,
        ],
        compiler_params=pltpu.CompilerParams(use_tc_tiling_on_sc=False,
                                             needs_layout_passes=False),
    )(idx_flat, _tc_linearize(table))

    return (out2.reshape(L_TOT, 4, BT, 8, 128)
            .transpose(2, 4, 1, 3, 0)
            .reshape(BATCH, EMBED_DIM, L_TOT))
